# baseline (speedup 1.0000x reference)
import jax
import jax.numpy as jnp
from jax import lax
from jax.experimental import pallas as pl
from jax.experimental.pallas import tpu as pltpu

OUT_D = 1000
B, L = 4096, 20
ROWS = B * L
BR = 1024
NBLK = ROWS // BR


def _body(x_ref, o_ref):
    iota = lax.broadcasted_iota(jnp.int32, (BR, OUT_D), 1)
    o_ref[...] = (iota == x_ref[...]).astype(jnp.float32)


def kernel(x):
    xf = x.reshape(ROWS, 1)
    out = pl.pallas_call(
        _body,
        grid=(NBLK,),
        in_specs=[pl.BlockSpec((BR, 1), lambda i: (i, 0))],
        out_specs=pl.BlockSpec((BR, OUT_D), lambda i: (i, 0)),
        out_shape=jax.ShapeDtypeStruct((ROWS, OUT_D), jnp.float32),
        compiler_params=pltpu.CompilerParams(
            dimension_semantics=("arbitrary",),
        ),
    )(xf)
    return out.reshape(B, L, OUT_D)
